# baseline (speedup 1.0000x reference)
import functools

import jax
import jax.numpy as jnp
from jax import lax
from jax.experimental import pallas as pl
from jax.experimental.pallas import tpu as pltpu
from jax.experimental.pallas import tpu_sc as plsc

N = 16384
DIM = 256
HEADS = 4
CODE_DIM = 256
K = 1024
IN_DIM = HEADS * CODE_DIM

BN = 512
NB = N // BN


def _codebook_proj_body(embed_ref, Wout_ref, bout_ref, embedP_ref, e2_ref):
    bout = bout_ref[...] * (1.0 / HEADS)
    for h in range(HEADS):
        e = embed_ref[h]
        w = Wout_ref[pl.ds(h * CODE_DIM, CODE_DIM), :]
        embedP_ref[h] = jnp.dot(e, w, preferred_element_type=jnp.float32) + bout
        e2_ref[h] = jnp.sum(e * e, axis=-1, keepdims=True).reshape(1, K)


def _distance_body(x_ref, Win_ref, bin_ref, embed_ref, e2_ref, ind_ref, loss_ref):
    i = pl.program_id(0)
    x = x_ref[...]
    xi = jnp.dot(x, Win_ref[...], preferred_element_type=jnp.float32) + bin_ref[...]

    partial = jnp.float32(0.0)
    for h in range(HEADS):
        xh = xi[:, h * CODE_DIM:(h + 1) * CODE_DIM]
        dots = lax.dot_general(
            xh, embed_ref[h], (((1,), (1,)), ((), ())),
            preferred_element_type=jnp.float32)
        x2 = jnp.sum(xh * xh, axis=1, keepdims=True)
        dist = -(x2 - 2.0 * dots + e2_ref[h])
        m = dist[:, :128]
        c = jnp.zeros((BN, 128), jnp.int32)
        for j in range(1, K // 128):
            dj = dist[:, j * 128:(j + 1) * 128]
            upd = dj > m
            m = jnp.where(upd, dj, m)
            c = jnp.where(upd, j, c)
        idx = c * 128 + lax.broadcasted_iota(jnp.int32, (BN, 128), 1)
        w = 128
        while w > 1:
            w //= 2
            m_lo, m_hi = m[:, :w], m[:, w:2 * w]
            i_lo, i_hi = idx[:, :w], idx[:, w:2 * w]
            upd = (m_hi > m_lo) | ((m_hi == m_lo) & (i_hi < i_lo))
            m = jnp.where(upd, m_hi, m_lo)
            idx = jnp.where(upd, i_hi, i_lo)
        ind_ref[0, h, :] = idx[:, 0] + h * K
        partial += jnp.sum(-m[:, 0])

    @pl.when(i == 0)
    def _():
        loss_ref[0, 0] = jnp.float32(0.0)

    loss_ref[0, 0] += partial

    @pl.when(i == pl.num_programs(0) - 1)
    def _():
        loss_ref[0, 0] = loss_ref[0, 0] * (1.0 / (HEADS * N * CODE_DIM))


def _codebook_proj(embed, W_out, b_out):
    return pl.pallas_call(
        _codebook_proj_body,
        out_shape=(
            jax.ShapeDtypeStruct((HEADS, K, DIM), jnp.float32),
            jax.ShapeDtypeStruct((HEADS, 1, K), jnp.float32),
        ),
    )(embed, W_out, b_out.reshape(1, DIM))


def _distances(x, W_in, b_in, embed, e2, nb):
    ind, loss = pl.pallas_call(
        _distance_body,
        grid=(nb,),
        in_specs=[
            pl.BlockSpec((BN, DIM), lambda i: (i, 0)),
            pl.BlockSpec((DIM, IN_DIM), lambda i: (0, 0)),
            pl.BlockSpec((1, IN_DIM), lambda i: (0, 0)),
            pl.BlockSpec((HEADS, K, CODE_DIM), lambda i: (0, 0, 0)),
            pl.BlockSpec((HEADS, 1, K), lambda i: (0, 0, 0)),
        ],
        out_specs=[
            pl.BlockSpec((1, HEADS, BN), lambda i: (i, 0, 0)),
            pl.BlockSpec((1, 1), lambda i: (0, 0), memory_space=pltpu.SMEM),
        ],
        out_shape=(
            jax.ShapeDtypeStruct((nb, HEADS, BN), jnp.int32),
            jax.ShapeDtypeStruct((1, 1), jnp.float32),
        ),
    )(x, W_in, b_in.reshape(1, IN_DIM), embed, e2)
    return ind, loss


def _gather_sum(embedP_flat, ind, n_rows):
    info = plsc.get_sparse_core_info()
    NC, NS = info.num_cores, info.num_subcores
    NW = NC * NS
    npw = n_rows // NW
    C = 32
    n_chunks = npw // C
    mesh = plsc.VectorSubcoreMesh(core_axis_name="c", subcore_axis_name="s")

    gbuf = pltpu.VMEM((C, DIM), jnp.float32)

    @functools.partial(
        pl.kernel,
        mesh=mesh,
        out_type=jax.ShapeDtypeStruct((n_rows, DIM), jnp.float32),
        scratch_types=[
            pltpu.VMEM((HEADS, npw), jnp.int32),
            [[gbuf] * HEADS, [gbuf] * HEADS],
            [gbuf, gbuf],
            [pltpu.SemaphoreType.DMA] * 2,
            [pltpu.SemaphoreType.DMA] * 2,
        ],
    )
    def gather_kernel(table_hbm, ind_hbm, out_hbm, idx_v, gsets, obufs,
                      gsems, osems):
        wid = lax.axis_index("s") * NC + lax.axis_index("c")
        base = wid * npw
        blk, off = (wid * npw) // BN, (wid * npw) % BN
        for h in range(HEADS):
            pltpu.sync_copy(ind_hbm.at[blk, h, pl.ds(off, npw)], idx_v.at[h])

        def issue_gathers(s, c):
            return [
                pltpu.async_copy(
                    table_hbm.at[idx_v.at[h, pl.ds(c * C, C)]],
                    gsets[s][h], gsems[s])
                for h in range(HEADS)
            ]

        pending = {0: issue_gathers(0, 0)}
        out_pending = {}
        for c in range(n_chunks):
            s = c % 2
            for cp in pending.pop(s):
                cp.wait()
            if c + 1 < n_chunks:
                pending[1 - s] = issue_gathers(1 - s, c + 1)
            if s in out_pending:
                out_pending.pop(s).wait()
            g0, g1, g2, g3 = gsets[s]
            ob = obufs[s]

            def row_body(j, carry, g0=g0, g1=g1, g2=g2, g3=g3, ob=ob):
                for i in range(DIM // 16):
                    sl = pl.ds(i * 16, 16)
                    ob[j, sl] = (g0[j, sl] + g1[j, sl]) + (g2[j, sl] + g3[j, sl])
                return carry

            lax.fori_loop(0, C, row_body, 0)
            out_pending[s] = pltpu.async_copy(
                ob, out_hbm.at[pl.ds(base + c * C, C)], osems[s])
        for cp in out_pending.values():
            cp.wait()

    return gather_kernel(embedP_flat, ind)


SPLIT = 4


def kernel(x, W_in, b_in, W_out, b_out, embed):
    embedP, e2 = _codebook_proj(embed, W_out, b_out)
    table = embedP.reshape(HEADS * K, DIM)
    ns = N // SPLIT
    outs, losses = [], []
    for s in range(SPLIT):
        ind_s, loss_s = _distances(
            x[s * ns:(s + 1) * ns], W_in, b_in, embed, e2, ns // BN)
        outs.append(_gather_sum(table, ind_s, ns))
        losses.append(loss_s.reshape(()))
    out = jnp.concatenate(outs, axis=0)
    loss = losses[0]
    for l_s in losses[1:]:
        loss = loss + l_s
    return out, loss


# speedup vs baseline: 1.1910x; 1.1910x over previous
import functools

import jax
import jax.numpy as jnp
from jax import lax
from jax.experimental import pallas as pl
from jax.experimental.pallas import tpu as pltpu
from jax.experimental.pallas import tpu_sc as plsc

N = 16384
DIM = 256
HEADS = 4
CODE_DIM = 256
K = 1024
IN_DIM = HEADS * CODE_DIM

BN = 512
NB = N // BN


def _codebook_proj_body(embed_ref, Wout_ref, bout_ref, embedP_ref, e2_ref):
    bout = bout_ref[...] * (1.0 / HEADS)
    for h in range(HEADS):
        e = embed_ref[h]
        w = Wout_ref[pl.ds(h * CODE_DIM, CODE_DIM), :]
        embedP_ref[h] = jnp.dot(e, w, preferred_element_type=jnp.float32) + bout
        e2_ref[h] = jnp.sum(e * e, axis=-1, keepdims=True).reshape(1, K)


def _distance_body(x_ref, Win_ref, bin_ref, embed_ref, e2_ref, ind_ref, loss_ref):
    i = pl.program_id(0)
    x = x_ref[...]
    xi = jnp.dot(x, Win_ref[...], preferred_element_type=jnp.float32) + bin_ref[...]

    partial = jnp.float32(0.0)
    for h in range(HEADS):
        xh = xi[:, h * CODE_DIM:(h + 1) * CODE_DIM]
        dots = lax.dot_general(
            xh, embed_ref[h], (((1,), (1,)), ((), ())),
            preferred_element_type=jnp.float32)
        x2 = jnp.sum(xh * xh, axis=1, keepdims=True)
        dist = -(x2 - 2.0 * dots + e2_ref[h])
        ind = jnp.argmax(dist, axis=1).astype(jnp.int32)
        maxv = jnp.max(dist, axis=1)
        ind_ref[0, h, :] = ind + h * K
        partial += jnp.sum(-maxv)

    @pl.when(i == 0)
    def _():
        loss_ref[0, 0] = jnp.float32(0.0)

    loss_ref[0, 0] += partial

    @pl.when(i == pl.num_programs(0) - 1)
    def _():
        loss_ref[0, 0] = loss_ref[0, 0] * (1.0 / (HEADS * N * CODE_DIM))


def _codebook_proj(embed, W_out, b_out):
    return pl.pallas_call(
        _codebook_proj_body,
        out_shape=(
            jax.ShapeDtypeStruct((HEADS, K, DIM), jnp.float32),
            jax.ShapeDtypeStruct((HEADS, 1, K), jnp.float32),
        ),
    )(embed, W_out, b_out.reshape(1, DIM))


def _distances(x, W_in, b_in, embed, e2, nb):
    ind, loss = pl.pallas_call(
        _distance_body,
        grid=(nb,),
        in_specs=[
            pl.BlockSpec((BN, DIM), lambda i: (i, 0)),
            pl.BlockSpec((DIM, IN_DIM), lambda i: (0, 0)),
            pl.BlockSpec((1, IN_DIM), lambda i: (0, 0)),
            pl.BlockSpec((HEADS, K, CODE_DIM), lambda i: (0, 0, 0)),
            pl.BlockSpec((HEADS, 1, K), lambda i: (0, 0, 0)),
        ],
        out_specs=[
            pl.BlockSpec((1, HEADS, BN), lambda i: (i, 0, 0)),
            pl.BlockSpec((1, 1), lambda i: (0, 0), memory_space=pltpu.SMEM),
        ],
        out_shape=(
            jax.ShapeDtypeStruct((nb, HEADS, BN), jnp.int32),
            jax.ShapeDtypeStruct((1, 1), jnp.float32),
        ),
    )(x, W_in, b_in.reshape(1, IN_DIM), embed, e2)
    return ind, loss


def _gather_sum(embedP_flat, ind, n_rows):
    info = plsc.get_sparse_core_info()
    NC, NS = info.num_cores, info.num_subcores
    NW = NC * NS
    npw = n_rows // NW
    C = 32
    n_chunks = npw // C
    mesh = plsc.VectorSubcoreMesh(core_axis_name="c", subcore_axis_name="s")

    gbuf = pltpu.VMEM((C, DIM), jnp.float32)

    @functools.partial(
        pl.kernel,
        mesh=mesh,
        out_type=jax.ShapeDtypeStruct((n_rows, DIM), jnp.float32),
        scratch_types=[
            pltpu.VMEM((HEADS, npw), jnp.int32),
            [[gbuf] * HEADS, [gbuf] * HEADS],
            [gbuf, gbuf],
            [pltpu.SemaphoreType.DMA] * 2,
            [pltpu.SemaphoreType.DMA] * 2,
        ],
    )
    def gather_kernel(table_hbm, ind_hbm, out_hbm, idx_v, gsets, obufs,
                      gsems, osems):
        wid = lax.axis_index("s") * NC + lax.axis_index("c")
        base = wid * npw
        blk, off = (wid * npw) // BN, (wid * npw) % BN
        for h in range(HEADS):
            pltpu.sync_copy(ind_hbm.at[blk, h, pl.ds(off, npw)], idx_v.at[h])

        def issue_gathers(s, c):
            return [
                pltpu.async_copy(
                    table_hbm.at[idx_v.at[h, pl.ds(c * C, C)]],
                    gsets[s][h], gsems[s])
                for h in range(HEADS)
            ]

        pending = {0: issue_gathers(0, 0)}
        out_pending = {}
        for c in range(n_chunks):
            s = c % 2
            for cp in pending.pop(s):
                cp.wait()
            if c + 1 < n_chunks:
                pending[1 - s] = issue_gathers(1 - s, c + 1)
            if s in out_pending:
                out_pending.pop(s).wait()
            g0, g1, g2, g3 = gsets[s]
            ob = obufs[s]

            def row_body(j, carry, g0=g0, g1=g1, g2=g2, g3=g3, ob=ob):
                for i in range(DIM // 16):
                    sl = pl.ds(i * 16, 16)
                    ob[j, sl] = (g0[j, sl] + g1[j, sl]) + (g2[j, sl] + g3[j, sl])
                return carry

            lax.fori_loop(0, C, row_body, 0)
            out_pending[s] = pltpu.async_copy(
                ob, out_hbm.at[pl.ds(base + c * C, C)], osems[s])
        for cp in out_pending.values():
            cp.wait()

    return gather_kernel(embedP_flat, ind)


SPLIT = 4


def kernel(x, W_in, b_in, W_out, b_out, embed):
    embedP, e2 = _codebook_proj(embed, W_out, b_out)
    table = embedP.reshape(HEADS * K, DIM)
    ns = N // SPLIT
    outs, losses = [], []
    for s in range(SPLIT):
        ind_s, loss_s = _distances(
            x[s * ns:(s + 1) * ns], W_in, b_in, embed, e2, ns // BN)
        outs.append(_gather_sum(table, ind_s, ns))
        losses.append(loss_s.reshape(()))
    out = jnp.concatenate(outs, axis=0)
    loss = losses[0]
    for l_s in losses[1:]:
        loss = loss + l_s
    return out, loss
